# initial kernel scaffold (unmeasured)
import jax
import jax.numpy as jnp
from jax import lax
from jax.experimental import pallas as pl
from jax.experimental.pallas import tpu as pltpu

N_DEV = 32
M = 4096
N = 8192
CM = M // N_DEV


def _allreduce_ring(partial):
    def body(p_ref, out_ref, acc, stage, recv_buf, send_sems, recv_sems,
             copy_in_sem, copy_out_sem, credit0, credit1):
        i = lax.axis_index("i")
        right = jnp.mod(i + 1, N_DEV)
        left = jnp.mod(i - 1, N_DEV)

        barrier = pltpu.get_barrier_semaphore()
        for nbr in (left, right):
            pl.semaphore_signal(barrier, inc=1, device_id=(nbr,),
                                device_id_type=pl.DeviceIdType.MESH)
        pl.semaphore_wait(barrier, 2)

        cp = pltpu.make_async_copy(p_ref.at[pl.ds(i * CM, CM)], acc,
                                   copy_in_sem)
        cp.start()
        cp.wait()

        for g in range(2 * (N_DEV - 1)):
            slot = g % 2
            credit = credit0 if slot == 0 else credit1
            if g >= 2:
                pl.semaphore_wait(credit, 1)

            rdma = pltpu.make_async_remote_copy(
                src_ref=acc,
                dst_ref=recv_buf.at[slot],
                send_sem=send_sems.at[slot],
                recv_sem=recv_sems.at[slot],
                device_id=(right,),
                device_id_type=pl.DeviceIdType.MESH,
            )
            rdma.start()

            if g < N_DEV - 1:
                rc = jnp.mod(i - g - 1, N_DEV)
                cp = pltpu.make_async_copy(p_ref.at[pl.ds(rc * CM, CM)],
                                           stage, copy_in_sem)
                cp.start()
                rdma.wait()
                cp.wait()
                acc[...] = stage[...] + recv_buf[slot]
            else:
                t = g - (N_DEV - 1)
                rc = jnp.mod(i - t, N_DEV)
                rdma.wait()
                acc[...] = recv_buf[slot]
                cpo = pltpu.make_async_copy(acc,
                                            out_ref.at[pl.ds(rc * CM, CM)],
                                            copy_out_sem)
                cpo.start()
                cpo.wait()

            pl.semaphore_signal(credit, inc=1, device_id=(left,),
                                device_id_type=pl.DeviceIdType.MESH)

            if g == N_DEV - 2:
                own = jnp.mod(i + 1, N_DEV)
                cpo = pltpu.make_async_copy(acc,
                                            out_ref.at[pl.ds(own * CM, CM)],
                                            copy_out_sem)
                cpo.start()
                cpo.wait()

        return

    return pl.pallas_call(
        body,
        out_shape=jax.ShapeDtypeStruct((M, N), jnp.float32),
        in_specs=[pl.BlockSpec(memory_space=pltpu.ANY)],
        out_specs=pl.BlockSpec(memory_space=pltpu.ANY),
        scratch_shapes=[
            pltpu.VMEM((CM, N), jnp.float32),
            pltpu.VMEM((CM, N), jnp.float32),
            pltpu.VMEM((2, CM, N), jnp.float32),
            pltpu.SemaphoreType.DMA((2,)),
            pltpu.SemaphoreType.DMA((2,)),
            pltpu.SemaphoreType.DMA,
            pltpu.SemaphoreType.DMA,
            pltpu.SemaphoreType.REGULAR,
            pltpu.SemaphoreType.REGULAR,
        ],
        compiler_params=pltpu.CompilerParams(collective_id=0),
    )(partial)


def kernel(x, w_mat):
    partial = jnp.dot(x, w_mat, preferred_element_type=jnp.float32)
    return _allreduce_ring(partial)


# baseline (device time: 3127365 ns/iter reference)
import jax
import jax.numpy as jnp
from jax import lax
from jax.experimental import pallas as pl
from jax.experimental.pallas import tpu as pltpu

N_DEV = 32
M = 4096
N = 8192
CM = M // N_DEV


def _allreduce_ring(partial):
    def body(p_ref, out_ref, acc, stage, recv_buf, send_sems, recv_sems,
             copy_in_sem, copy_out_sem, credit0, credit1):
        i = lax.axis_index("i")
        right = jnp.mod(i + 1, N_DEV)
        left = jnp.mod(i - 1, N_DEV)

        barrier = pltpu.get_barrier_semaphore()
        for nbr in (left, right):
            pl.semaphore_signal(barrier, inc=1, device_id=(nbr,),
                                device_id_type=pl.DeviceIdType.MESH)
        pl.semaphore_wait(barrier, 2)

        cp = pltpu.make_async_copy(p_ref.at[pl.ds(i * CM, CM)], acc,
                                   copy_in_sem)
        cp.start()
        cp.wait()

        for g in range(2 * (N_DEV - 1)):
            slot = g % 2
            credit = credit0 if slot == 0 else credit1
            if g >= 2:
                pl.semaphore_wait(credit, 1)

            rdma = pltpu.make_async_remote_copy(
                src_ref=acc,
                dst_ref=recv_buf.at[slot],
                send_sem=send_sems.at[slot],
                recv_sem=recv_sems.at[slot],
                device_id=(right,),
                device_id_type=pl.DeviceIdType.MESH,
            )
            rdma.start()

            if g < N_DEV - 1:
                rc = jnp.mod(i - g - 1, N_DEV)
                cp = pltpu.make_async_copy(p_ref.at[pl.ds(rc * CM, CM)],
                                           stage, copy_in_sem)
                cp.start()
                rdma.wait()
                cp.wait()
                acc[...] = stage[...] + recv_buf[slot]
            else:
                t = g - (N_DEV - 1)
                rc = jnp.mod(i - t, N_DEV)
                rdma.wait()
                acc[...] = recv_buf[slot]
                cpo = pltpu.make_async_copy(acc,
                                            out_ref.at[pl.ds(rc * CM, CM)],
                                            copy_out_sem)
                cpo.start()
                cpo.wait()

            if g < 2 * (N_DEV - 1) - 2:
                pl.semaphore_signal(credit, inc=1, device_id=(left,),
                                    device_id_type=pl.DeviceIdType.MESH)

            if g == N_DEV - 2:
                own = jnp.mod(i + 1, N_DEV)
                cpo = pltpu.make_async_copy(acc,
                                            out_ref.at[pl.ds(own * CM, CM)],
                                            copy_out_sem)
                cpo.start()
                cpo.wait()

        return

    return pl.pallas_call(
        body,
        out_shape=jax.ShapeDtypeStruct((M, N), jnp.float32),
        in_specs=[pl.BlockSpec(memory_space=pl.ANY)],
        out_specs=pl.BlockSpec(memory_space=pl.ANY),
        scratch_shapes=[
            pltpu.VMEM((CM, N), jnp.float32),
            pltpu.VMEM((CM, N), jnp.float32),
            pltpu.VMEM((2, CM, N), jnp.float32),
            pltpu.SemaphoreType.DMA((2,)),
            pltpu.SemaphoreType.DMA((2,)),
            pltpu.SemaphoreType.DMA,
            pltpu.SemaphoreType.DMA,
            pltpu.SemaphoreType.REGULAR,
            pltpu.SemaphoreType.REGULAR,
        ],
        compiler_params=pltpu.CompilerParams(collective_id=0),
    )(partial)


def kernel(x, w_mat):
    partial = jnp.dot(x, w_mat, preferred_element_type=jnp.float32)
    return _allreduce_ring(partial)


# device time: 3109595 ns/iter; 1.0057x vs baseline; 1.0057x over previous
import jax
import jax.numpy as jnp
from jax import lax
from jax.experimental import pallas as pl
from jax.experimental.pallas import tpu as pltpu

N_DEV = 32
M = 4096
N = 8192
CM = M // N_DEV
HM = CM // 2
N_STEPS = 2 * (N_DEV - 1)


def _allreduce_ring(partial):
    def body(p_ref, out_ref, acc_a, acc_b, stage_a, stage_b, recv_a, recv_b,
             send_sems_a, recv_sems_a, send_sems_b, recv_sems_b,
             copy_in_sem_a, copy_in_sem_b, copy_out_sem_a, copy_out_sem_b,
             ca0, ca1, cb0, cb1):
        i = lax.axis_index("i")
        right = jnp.mod(i + 1, N_DEV)
        left = jnp.mod(i - 1, N_DEV)

        barrier = pltpu.get_barrier_semaphore()
        for nbr in (left, right):
            pl.semaphore_signal(barrier, inc=1, device_id=(nbr,),
                                device_id_type=pl.DeviceIdType.MESH)
        pl.semaphore_wait(barrier, 2)

        cp_a = pltpu.make_async_copy(p_ref.at[pl.ds(i * CM, HM)], acc_a,
                                     copy_in_sem_a)
        cp_b = pltpu.make_async_copy(p_ref.at[pl.ds(i * CM + HM, HM)], acc_b,
                                     copy_in_sem_b)
        cp_a.start()
        cp_b.start()
        cp_a.wait()
        cp_b.wait()

        for g in range(N_STEPS):
            slot = g % 2
            credit_a = ca0 if slot == 0 else ca1
            credit_b = cb0 if slot == 0 else cb1
            if g >= 2:
                pl.semaphore_wait(credit_a, 1)
                pl.semaphore_wait(credit_b, 1)

            rdma_a = pltpu.make_async_remote_copy(
                src_ref=acc_a,
                dst_ref=recv_a.at[slot],
                send_sem=send_sems_a.at[slot],
                recv_sem=recv_sems_a.at[slot],
                device_id=(right,),
                device_id_type=pl.DeviceIdType.MESH,
            )
            rdma_b = pltpu.make_async_remote_copy(
                src_ref=acc_b,
                dst_ref=recv_b.at[slot],
                send_sem=send_sems_b.at[slot],
                recv_sem=recv_sems_b.at[slot],
                device_id=(left,),
                device_id_type=pl.DeviceIdType.MESH,
            )
            rdma_a.start()
            rdma_b.start()

            if g < N_DEV - 1:
                rc_a = jnp.mod(i - g - 1, N_DEV)
                rc_b = jnp.mod(i + g + 1, N_DEV)
                cp_a = pltpu.make_async_copy(
                    p_ref.at[pl.ds(rc_a * CM, HM)], stage_a, copy_in_sem_a)
                cp_b = pltpu.make_async_copy(
                    p_ref.at[pl.ds(rc_b * CM + HM, HM)], stage_b,
                    copy_in_sem_b)
                cp_a.start()
                cp_b.start()
                rdma_a.wait()
                rdma_b.wait()
                cp_a.wait()
                cp_b.wait()
                acc_a[...] = stage_a[...] + recv_a[slot]
                acc_b[...] = stage_b[...] + recv_b[slot]
            else:
                t = g - (N_DEV - 1)
                rc_a = jnp.mod(i - t, N_DEV)
                rc_b = jnp.mod(i + t, N_DEV)
                rdma_a.wait()
                rdma_b.wait()
                acc_a[...] = recv_a[slot]
                acc_b[...] = recv_b[slot]
                cpo_a = pltpu.make_async_copy(
                    acc_a, out_ref.at[pl.ds(rc_a * CM, HM)], copy_out_sem_a)
                cpo_b = pltpu.make_async_copy(
                    acc_b, out_ref.at[pl.ds(rc_b * CM + HM, HM)],
                    copy_out_sem_b)
                cpo_a.start()
                cpo_b.start()
                cpo_a.wait()
                cpo_b.wait()

            if g < N_STEPS - 2:
                pl.semaphore_signal(credit_a, inc=1, device_id=(left,),
                                    device_id_type=pl.DeviceIdType.MESH)
                pl.semaphore_signal(credit_b, inc=1, device_id=(right,),
                                    device_id_type=pl.DeviceIdType.MESH)

            if g == N_DEV - 2:
                own_a = jnp.mod(i + 1, N_DEV)
                own_b = jnp.mod(i - 1, N_DEV)
                cpo_a = pltpu.make_async_copy(
                    acc_a, out_ref.at[pl.ds(own_a * CM, HM)], copy_out_sem_a)
                cpo_b = pltpu.make_async_copy(
                    acc_b, out_ref.at[pl.ds(own_b * CM + HM, HM)],
                    copy_out_sem_b)
                cpo_a.start()
                cpo_b.start()
                cpo_a.wait()
                cpo_b.wait()

    return pl.pallas_call(
        body,
        out_shape=jax.ShapeDtypeStruct((M, N), jnp.float32),
        in_specs=[pl.BlockSpec(memory_space=pl.ANY)],
        out_specs=pl.BlockSpec(memory_space=pl.ANY),
        scratch_shapes=[
            pltpu.VMEM((HM, N), jnp.float32),
            pltpu.VMEM((HM, N), jnp.float32),
            pltpu.VMEM((HM, N), jnp.float32),
            pltpu.VMEM((HM, N), jnp.float32),
            pltpu.VMEM((2, HM, N), jnp.float32),
            pltpu.VMEM((2, HM, N), jnp.float32),
            pltpu.SemaphoreType.DMA((2,)),
            pltpu.SemaphoreType.DMA((2,)),
            pltpu.SemaphoreType.DMA((2,)),
            pltpu.SemaphoreType.DMA((2,)),
            pltpu.SemaphoreType.DMA,
            pltpu.SemaphoreType.DMA,
            pltpu.SemaphoreType.DMA,
            pltpu.SemaphoreType.DMA,
            pltpu.SemaphoreType.REGULAR,
            pltpu.SemaphoreType.REGULAR,
            pltpu.SemaphoreType.REGULAR,
            pltpu.SemaphoreType.REGULAR,
        ],
        compiler_params=pltpu.CompilerParams(collective_id=0),
    )(partial)


def kernel(x, w_mat):
    partial = jnp.dot(x, w_mat, preferred_element_type=jnp.float32)
    return _allreduce_ring(partial)
